# baseline (speedup 1.0000x reference)
import functools

import jax
import jax.numpy as jnp
from jax import lax
from jax.experimental import pallas as pl
from jax.experimental.pallas import tpu as pltpu
from jax.experimental.pallas import tpu_sc as plsc

_NW = 32
_DEPTH = 8


def _gather_kernel(bsz, seq, vocab, dim, xt_hbm, table_hbm, out_hbm,
                   idx_v, *bufs):
    rows = bufs[0:_DEPTH]
    trows = bufs[_DEPTH:2 * _DEPTH]
    semg = bufs[2 * _DEPTH:3 * _DEPTH]
    semw = bufs[3 * _DEPTH:4 * _DEPTH]

    wid = lax.axis_index("s") * 2 + lax.axis_index("c")
    bw = bsz // _NW
    b0 = wid * bw
    nb = bsz // 128
    ng = dim // 8

    pltpu.sync_copy(xt_hbm.at[:, pl.ds(b0, bw)], idx_v)

    lanes = lax.iota(jnp.int32, 16)

    def g_desc(s, k):
        return pltpu.make_async_copy(table_hbm.at[idx_v.at[s]], rows[k],
                                     semg[k])

    def transpose_block(k):
        for c in range(dim):
            col = jnp.full((16,), c, jnp.int32)
            for i0 in range(0, bw, 16):
                v = plsc.load_gather(rows[k], [lanes + i0, col])
                trows[k][pl.ds(c * 128 + i0, 16)] = v

    def w_descs(s, k):
        return [
            pltpu.make_async_copy(
                trows[k].at[pl.ds(g * 1024, 1024)],
                out_hbm.at[pl.ds(((s * ng + g) * nb + wid) * 1024, 1024)],
                semw[k])
            for g in range(ng)
        ]

    n_iters = seq // _DEPTH

    for k in range(_DEPTH):
        g_desc(k, k).start()

    def body(j, _):
        s0 = j * _DEPTH
        for k in range(_DEPTH):
            g_desc(s0 + k, k).wait()

            @pl.when(j > 0)
            def _wb_wait():
                for w in w_descs(s0 - _DEPTH + k, k):
                    w.wait()

            transpose_block(k)

            @pl.when(j < n_iters - 1)
            def _refill():
                g_desc(s0 + _DEPTH + k, k).start()

            for w in w_descs(s0 + k, k):
                w.start()
        return 0

    lax.fori_loop(0, n_iters, body, 0)

    for k in range(_DEPTH):
        for w in w_descs(seq - _DEPTH + k, k):
            w.wait()


def kernel(x, embed):
    b, s = x.shape
    v, d = embed.shape
    nb = b // 128

    mesh = plsc.VectorSubcoreMesh(core_axis_name="c", subcore_axis_name="s")

    run = pl.kernel(
        functools.partial(_gather_kernel, b, s, v, d),
        mesh=mesh,
        out_type=jax.ShapeDtypeStruct((s * (d // 8) * nb * 1024,),
                                      jnp.float32),
        scratch_types=(
            [pltpu.VMEM((s, b // _NW), jnp.int32)]
            + [pltpu.VMEM((b // _NW, d), jnp.float32)] * _DEPTH
            + [pltpu.VMEM((b // _NW * d,), jnp.float32)] * _DEPTH
            + [pltpu.SemaphoreType.DMA] * (2 * _DEPTH)
        ),
        compiler_params=pltpu.CompilerParams(use_tc_tiling_on_sc=False,
                                             needs_layout_passes=False),
    )
    xt = jnp.transpose(x.astype(jnp.int32))
    out1 = run(xt, embed)
    out5 = out1.reshape(s, d // 8, nb, 8, 128)
    return out5.transpose(2, 4, 0, 1, 3).reshape(b, s, d)


# speedup vs baseline: 1.0014x; 1.0014x over previous
import functools

import jax
import jax.numpy as jnp
from jax import lax
from jax.experimental import pallas as pl
from jax.experimental.pallas import tpu as pltpu
from jax.experimental.pallas import tpu_sc as plsc

_NW = 32
_DEPTH = 10


def _gather_kernel(bsz, seq, vocab, dim, xt_hbm, table_hbm, out_hbm,
                   idx_v, *bufs):
    rows = bufs[0:_DEPTH]
    trows = bufs[_DEPTH:2 * _DEPTH]
    semg = bufs[2 * _DEPTH:3 * _DEPTH]
    semw = bufs[3 * _DEPTH:4 * _DEPTH]

    wid = lax.axis_index("s") * 2 + lax.axis_index("c")
    bw = bsz // _NW
    b0 = wid * bw
    nb = bsz // 128
    ng = dim // 8

    pltpu.sync_copy(xt_hbm.at[:, pl.ds(b0, bw)], idx_v)

    lanes = lax.iota(jnp.int32, 16)

    def g_desc(s, k):
        return pltpu.make_async_copy(table_hbm.at[idx_v.at[s]], rows[k],
                                     semg[k])

    def transpose_block(k):
        for c in range(dim):
            col = jnp.full((16,), c, jnp.int32)
            for i0 in range(0, bw, 16):
                v = plsc.load_gather(rows[k], [lanes + i0, col])
                trows[k][pl.ds(c * 128 + i0, 16)] = v

    def w_descs(s, k):
        return [
            pltpu.make_async_copy(
                trows[k].at[pl.ds(g * 1024, 1024)],
                out_hbm.at[pl.ds(((s * ng + g) * nb + wid) * 1024, 1024)],
                semw[k])
            for g in range(ng)
        ]

    n_iters = seq // _DEPTH

    for k in range(_DEPTH):
        g_desc(k, k).start()

    def body(j, _):
        s0 = j * _DEPTH
        for k in range(_DEPTH):
            g_desc(s0 + k, k).wait()

            @pl.when(j > 0)
            def _wb_wait():
                for w in w_descs(s0 - _DEPTH + k, k):
                    w.wait()

            transpose_block(k)

            @pl.when(j < n_iters - 1)
            def _refill():
                g_desc(s0 + _DEPTH + k, k).start()

            for w in w_descs(s0 + k, k):
                w.start()
        return 0

    lax.fori_loop(0, n_iters, body, 0)

    for k in range(_DEPTH):
        for w in w_descs(seq - _DEPTH + k, k):
            w.wait()


def kernel(x, embed):
    b, s = x.shape
    v, d = embed.shape
    nb = b // 128

    mesh = plsc.VectorSubcoreMesh(core_axis_name="c", subcore_axis_name="s")

    run = pl.kernel(
        functools.partial(_gather_kernel, b, s, v, d),
        mesh=mesh,
        out_type=jax.ShapeDtypeStruct((s * (d // 8) * nb * 1024,),
                                      jnp.float32),
        scratch_types=(
            [pltpu.VMEM((s, b // _NW), jnp.int32)]
            + [pltpu.VMEM((b // _NW, d), jnp.float32)] * _DEPTH
            + [pltpu.VMEM((b // _NW * d,), jnp.float32)] * _DEPTH
            + [pltpu.SemaphoreType.DMA] * (2 * _DEPTH)
        ),
        compiler_params=pltpu.CompilerParams(use_tc_tiling_on_sc=False,
                                             needs_layout_passes=False),
    )
    xt = jnp.transpose(x.astype(jnp.int32))
    out1 = run(xt, embed)
    out5 = out1.reshape(s, d // 8, nb, 8, 128)
    return out5.transpose(2, 4, 0, 1, 3).reshape(b, s, d)
